# baseline (speedup 1.0000x reference)
import functools

import jax
import jax.numpy as jnp
from jax import lax
from jax.experimental import pallas as pl
from jax.experimental.pallas import tpu as pltpu
from jax.experimental.pallas import tpu_sc as plsc

NUM_ROWS = 1000000
BATCH = 16384
D = 30

_info = plsc.get_sparse_core_info()
NC = _info.num_cores
NS = _info.num_subcores
L = _info.num_lanes
NW = NC * NS
BPW = BATCH // NW
CH = 128
NCHUNK = BPW // CH

_mesh = plsc.VectorSubcoreMesh(core_axis_name="c", subcore_axis_name="s")


@functools.partial(
    pl.kernel,
    mesh=_mesh,
    out_type=jax.ShapeDtypeStruct((BATCH,), jnp.float32),
    compiler_params=pltpu.CompilerParams(
        use_tc_tiling_on_sc=False,
        needs_layout_passes=False,
    ),
    scratch_types=[
        pltpu.VMEM((NCHUNK, CH), jnp.int32),
        pltpu.VMEM((NCHUNK, CH), jnp.int32),
        pltpu.VMEM((D, BPW), jnp.float32),
        pltpu.VMEM((D, BPW), jnp.float32),
        pltpu.VMEM((BPW,), jnp.float32),
        pltpu.SemaphoreType.DMA,
    ],
)
def _recommender_sc(uidx_hbm, iidx_hbm, wt_hbm, xt_hbm, out_hbm,
                    uidx_v, iidx_v, wg, xg, out_v, sem):
    wid = lax.axis_index("s") * NC + lax.axis_index("c")
    base = wid * BPW

    pltpu.sync_copy(uidx_hbm.at[wid], uidx_v)
    pltpu.sync_copy(iidx_hbm.at[wid], iidx_v)

    copies = []
    for d in range(D):
        for j in range(NCHUNK):
            dsl = pl.ds(j * CH, CH)
            copies.append(pltpu.async_copy(
                wt_hbm.at[d].at[uidx_v.at[j]], wg.at[d, dsl], sem))
            copies.append(pltpu.async_copy(
                xt_hbm.at[d].at[iidx_v.at[j]], xg.at[d, dsl], sem))
    for c in copies:
        c.wait()

    def group_body(g, carry):
        gs = pl.ds(g * L, L)
        acc = jnp.zeros((L,), jnp.float32)
        for d in range(D):
            acc = acc + wg[d, gs] * xg[d, gs]
        out_v[gs] = acc
        return carry

    lax.fori_loop(0, BPW // L, group_body, 0)

    pltpu.sync_copy(out_v, out_hbm.at[pl.ds(base, BPW)])


def kernel(x, W, X):
    uidx = x[:, 0].astype(jnp.int32).reshape(NW, NCHUNK, CH)
    iidx = x[:, 1].astype(jnp.int32).reshape(NW, NCHUNK, CH)
    return _recommender_sc(uidx, iidx, W.T, X.T)


# speedup vs baseline: 3.9908x; 3.9908x over previous
import functools

import jax
import jax.numpy as jnp
from jax import lax
from jax.experimental import pallas as pl
from jax.experimental.pallas import tpu as pltpu
from jax.experimental.pallas import tpu_sc as plsc

NUM_ROWS = 1000000
BATCH = 16384
D = 30
ROW32 = 32
NR32 = NUM_ROWS * D // ROW32
MAXR = NR32 - 1

_info = plsc.get_sparse_core_info()
NC = _info.num_cores
NS = _info.num_subcores
L = _info.num_lanes
NW = NC * NS
BPW = BATCH // NW
IDX_CHUNK = 128
NCHUNK = BPW // IDX_CHUNK

_mesh = plsc.VectorSubcoreMesh(core_axis_name="c", subcore_axis_name="s")


@functools.partial(
    pl.kernel,
    mesh=_mesh,
    out_type=jax.ShapeDtypeStruct((BATCH,), jnp.float32),
    compiler_params=pltpu.CompilerParams(
        use_tc_tiling_on_sc=False,
        needs_layout_passes=False,
    ),
    scratch_types=[
        pltpu.VMEM((BPW,), jnp.int32),
        pltpu.VMEM((BPW,), jnp.int32),
        pltpu.VMEM((NCHUNK, IDX_CHUNK), jnp.int32),
        pltpu.VMEM((NCHUNK, IDX_CHUNK), jnp.int32),
        pltpu.VMEM((NCHUNK, IDX_CHUNK), jnp.int32),
        pltpu.VMEM((NCHUNK, IDX_CHUNK), jnp.int32),
        pltpu.VMEM((2 * BPW, ROW32), jnp.float32),
        pltpu.VMEM((2 * BPW, ROW32), jnp.float32),
        pltpu.VMEM((BPW,), jnp.float32),
        pltpu.SemaphoreType.DMA,
    ],
)
def _recommender_sc(uraw_hbm, iraw_hbm, w_hbm, x_hbm, out_hbm,
                    uraw_v, iraw_v, ua_v, ub_v, xa_v, xb_v,
                    wbuf, xbuf, out_v, sem):
    wid = lax.axis_index("s") * NC + lax.axis_index("c")
    base = wid * BPW

    pltpu.sync_copy(uraw_hbm.at[wid], uraw_v)
    pltpu.sync_copy(iraw_hbm.at[wid], iraw_v)

    for j in range(NCHUNK):
        for k in range(IDX_CHUNK // L):
            sl = pl.ds(j * IDX_CHUNK + k * L, L)
            dsl = pl.ds(k * L, L)
            for raw_v, a_v, b_v in ((uraw_v, ua_v, ub_v),
                                    (iraw_v, xa_v, xb_v)):
                t = raw_v[sl] * D
                a = t >> 5
                a_v[j, dsl] = a
                b_v[j, dsl] = jnp.minimum(a + 1, MAXR)

    copies = []
    for j in range(NCHUNK):
        losl = pl.ds(j * IDX_CHUNK, IDX_CHUNK)
        hisl = pl.ds(BPW + j * IDX_CHUNK, IDX_CHUNK)
        copies.append(pltpu.async_copy(w_hbm.at[ua_v.at[j]], wbuf.at[losl], sem))
        copies.append(pltpu.async_copy(w_hbm.at[ub_v.at[j]], wbuf.at[hisl], sem))
        copies.append(pltpu.async_copy(x_hbm.at[xa_v.at[j]], xbuf.at[losl], sem))
        copies.append(pltpu.async_copy(x_hbm.at[xb_v.at[j]], xbuf.at[hisl], sem))
    for c in copies:
        c.wait()

    lane = lax.iota(jnp.int32, L)

    def group_body(g, carry):
        gs = pl.ds(g * L, L)
        rows = g * L + lane
        uo = (uraw_v[gs] * D) & 31
        io = (iraw_v[gs] * D) & 31
        acc = jnp.zeros((L,), jnp.float32)
        for d in range(D):
            uw = uo + d
            iw = io + d
            wv = plsc.load_gather(wbuf, [rows + ((uw >> 5) << 9), uw & 31])
            xv = plsc.load_gather(xbuf, [rows + ((iw >> 5) << 9), iw & 31])
            acc = acc + wv * xv
        out_v[gs] = acc
        return carry

    lax.fori_loop(0, BPW // L, group_body, 0)

    pltpu.sync_copy(out_v, out_hbm.at[pl.ds(base, BPW)])


def kernel(x, W, X):
    uraw = x[:, 0].astype(jnp.int32).reshape(NW, BPW)
    iraw = x[:, 1].astype(jnp.int32).reshape(NW, BPW)
    w32 = W.reshape(NR32, ROW32)
    x32 = X.reshape(NR32, ROW32)
    return _recommender_sc(uraw, iraw, w32, x32)
